# Initial kernel scaffold; baseline (speedup 1.0000x reference)
import jax
import jax.numpy as jnp
from jax.experimental import pallas as pl


def kernel(x, table):
    raise NotImplementedError("write your pallas kernel here")


# speedup vs baseline: 2.9496x; 2.9496x over previous
import functools

import jax
import jax.numpy as jnp
import numpy as np
from jax import lax
from jax.experimental import pallas as pl
from jax.experimental.pallas import tpu as pltpu
from jax.experimental.pallas import tpu_sc as plsc

_FIELD_DIMS = [1000] * 26
_NUM_FIELDS = len(_FIELD_DIMS)
_EMBED = 128
_BATCH = 16384
_OFFSETS = np.array((0, *np.cumsum(_FIELD_DIMS)[:-1]), dtype=np.int32)

_N = _BATCH * _NUM_FIELDS
_NW = 32
_PER_W = _N // _NW
_CHUNK = 128
_NCHUNK = _PER_W // _CHUNK
_LANES = 16

_OFFS_TILED = np.tile(_OFFSETS, _PER_W // _NUM_FIELDS).reshape(_NCHUNK, _CHUNK)

_mesh = plsc.VectorSubcoreMesh(core_axis_name="c", subcore_axis_name="s")


@functools.partial(
    pl.kernel,
    mesh=_mesh,
    out_type=jax.ShapeDtypeStruct((_N, _EMBED), jnp.float32),
    scratch_types=[
        pltpu.VMEM((_NCHUNK, _CHUNK), jnp.int32),
        pltpu.VMEM((_NCHUNK, _CHUNK), jnp.int32),
        pltpu.VMEM((2, _CHUNK, _EMBED), jnp.float32),
        pltpu.SemaphoreType.DMA,
        pltpu.SemaphoreType.DMA,
    ],
)
def _emb_lookup(x_hbm, offs_hbm, table_hbm, out_hbm, idx_v, offs_v, rows_v,
                gsem, wsem):
    wid = lax.axis_index("s") * 2 + lax.axis_index("c")
    row_base = wid * _PER_W

    pltpu.sync_copy(x_hbm.at[wid], idx_v)
    pltpu.sync_copy(offs_hbm, offs_v)

    def _add_row(j, carry):
        for v in range(_CHUNK // _LANES):
            sl = pl.ds(v * _LANES, _LANES)
            idx_v[j, sl] = idx_v[j, sl] + offs_v[j, sl]
        return carry

    lax.fori_loop(0, _NCHUNK, _add_row, 0)

    def _chunk(j, carry):
        pltpu.async_copy(table_hbm.at[idx_v.at[j]], rows_v.at[0], gsem).wait()
        pltpu.async_copy(
            rows_v.at[0], out_hbm.at[pl.ds(row_base + j * _CHUNK, _CHUNK)],
            wsem).wait()
        return carry

    lax.fori_loop(0, _NCHUNK, _chunk, 0)


def kernel(x, table):
    x_blocks = x.astype(jnp.int32).reshape(_NW, _NCHUNK, _CHUNK)
    offs = jnp.asarray(_OFFS_TILED, dtype=jnp.int32)
    out = _emb_lookup(x_blocks, offs, table)
    return out.reshape(_BATCH, _NUM_FIELDS, _EMBED)


# speedup vs baseline: 3.3384x; 1.1318x over previous
import functools

import jax
import jax.numpy as jnp
import numpy as np
from jax import lax
from jax.experimental import pallas as pl
from jax.experimental.pallas import tpu as pltpu
from jax.experimental.pallas import tpu_sc as plsc

_FIELD_DIMS = [1000] * 26
_NUM_FIELDS = len(_FIELD_DIMS)
_EMBED = 128
_BATCH = 16384
_OFFSETS = np.array((0, *np.cumsum(_FIELD_DIMS)[:-1]), dtype=np.int32)

_N = _BATCH * _NUM_FIELDS
_NW = 32
_PER_W = _N // _NW
_CHUNK = 128
_NCHUNK = _PER_W // _CHUNK
_LANES = 16

_OFFS_TILED = np.tile(_OFFSETS, _PER_W // _NUM_FIELDS).reshape(_NCHUNK, _CHUNK)

_mesh = plsc.VectorSubcoreMesh(core_axis_name="c", subcore_axis_name="s")


@functools.partial(
    pl.kernel,
    mesh=_mesh,
    out_type=jax.ShapeDtypeStruct((_N, _EMBED), jnp.float32),
    scratch_types=[
        pltpu.VMEM((_NCHUNK, _CHUNK), jnp.int32),
        pltpu.VMEM((_NCHUNK, _CHUNK), jnp.int32),
        pltpu.VMEM((4, _CHUNK, _EMBED), jnp.float32),
        pltpu.SemaphoreType.DMA,
        pltpu.SemaphoreType.DMA,
    ],
)
def _emb_lookup(x_hbm, offs_hbm, table_hbm, out_hbm, idx_v, offs_v, rows_v,
                gsem, wsem):
    wid = lax.axis_index("s") * 2 + lax.axis_index("c")
    row_base = wid * _PER_W

    pltpu.sync_copy(x_hbm.at[wid], idx_v)
    pltpu.sync_copy(offs_hbm, offs_v)

    def _add_row(j, carry):
        for v in range(_CHUNK // _LANES):
            sl = pl.ds(v * _LANES, _LANES)
            idx_v[j, sl] = idx_v[j, sl] + offs_v[j, sl]
        return carry

    lax.fori_loop(0, _NCHUNK, _add_row, 0)

    def _start_gather(j, b):
        pltpu.async_copy(table_hbm.at[idx_v.at[j]], rows_v.at[b], gsem)

    def _wait_gather(j, b):
        pltpu.make_async_copy(table_hbm.at[idx_v.at[j]], rows_v.at[b],
                              gsem).wait()

    def _out_slice(j):
        return out_hbm.at[pl.ds(row_base + j * _CHUNK, _CHUNK)]

    def _start_write(j, b):
        pltpu.async_copy(rows_v.at[b], _out_slice(j), wsem)

    def _wait_write(j, b):
        pltpu.make_async_copy(rows_v.at[b], _out_slice(j), wsem).wait()

    _start_gather(0, 0)
    _start_gather(1, 1)
    for j in (0, 1):
        _wait_gather(j, j)
        _start_write(j, j)
        _start_gather(j + 2, j + 2)

    def _steady(s, carry):
        for k in range(4):
            j = 2 + s * 4 + k
            b = (2 + k) % 4
            _wait_gather(j, b)
            _start_write(j, b)
            _wait_write(j - 2, (b + 2) % 4)
            _start_gather(j + 2, (b + 2) % 4)
        return carry

    lax.fori_loop(0, (_NCHUNK - 4) // 4, _steady, 0)

    for j in (_NCHUNK - 2, _NCHUNK - 1):
        b = j % 4
        _wait_gather(j, b)
        _start_write(j, b)
        _wait_write(j - 2, (b + 2) % 4)
    for j in (_NCHUNK - 2, _NCHUNK - 1):
        _wait_write(j, j % 4)


def kernel(x, table):
    x_blocks = x.astype(jnp.int32).reshape(_NW, _NCHUNK, _CHUNK)
    offs = jnp.asarray(_OFFS_TILED, dtype=jnp.int32)
    out = _emb_lookup(x_blocks, offs, table)
    return out.reshape(_BATCH, _NUM_FIELDS, _EMBED)
